# baseline (speedup 1.0000x reference)
import functools

import jax
import jax.numpy as jnp
from jax.experimental import pallas as pl
from jax.experimental.pallas import tpu as pltpu
from jax.experimental.pallas import tpu_sc as plsc

NUM_EMBEDDINGS = 1024
EMBEDDING_DIM = 64
COMMITMENT_COST = 0.25


_BM = 1024


def _dist_argmin_body(x_ref, w_ref, idx_ref, loss_ref):
    x = x_ref[...]
    w = w_ref[...]
    m = jax.lax.dot_general(x, w, (((1,), (1,)), ((), ())),
                            preferred_element_type=jnp.float32)
    x2 = jnp.sum(x * x, axis=1, keepdims=True)
    w2 = jnp.sum(w * w, axis=1)
    dist = (x2 - 2.0 * m) + w2[None, :]
    minval = jnp.min(dist, axis=1, keepdims=True)
    iota = jax.lax.broadcasted_iota(jnp.int32, dist.shape, 1)
    cand = jnp.where(dist == minval, iota, jnp.int32(NUM_EMBEDDINGS))
    idx_ref[...] = jnp.min(cand, axis=1)

    @pl.when(pl.program_id(0) == 0)
    def _init():
        loss_ref[...] = jnp.zeros_like(loss_ref)

    loss_ref[...] += jnp.sum(minval).reshape(1, 1)


def _dist_argmin(flat, W):
    n = flat.shape[0]
    grid = (n // _BM,)
    idx, loss = pl.pallas_call(
        _dist_argmin_body,
        grid=grid,
        in_specs=[
            pl.BlockSpec((_BM, EMBEDDING_DIM), lambda i: (i, 0)),
            pl.BlockSpec((NUM_EMBEDDINGS, EMBEDDING_DIM), lambda i: (0, 0)),
        ],
        out_specs=[
            pl.BlockSpec((_BM,), lambda i: (i,)),
            pl.BlockSpec((1, 1), lambda i: (0, 0)),
        ],
        out_shape=[
            jax.ShapeDtypeStruct((n,), jnp.int32),
            jax.ShapeDtypeStruct((1, 1), jnp.float32),
        ],
    )(flat, W)
    return idx, loss[0, 0]


_NW = 32
_CHUNK = 128
_ROW = 128


def _gather_rows(W_pad, idx):
    n = idx.shape[0]
    b_per_w = n // _NW
    nch = b_per_w // _CHUNK
    idx3 = idx.reshape(_NW, nch, _CHUNK)
    mesh = plsc.VectorSubcoreMesh(core_axis_name="c", subcore_axis_name="s")

    @functools.partial(
        pl.kernel,
        mesh=mesh,
        out_type=jax.ShapeDtypeStruct((n, _ROW), jnp.float32),
        scratch_types=[
            pltpu.VMEM((nch, _CHUNK), jnp.int32),
            pltpu.VMEM((b_per_w, _ROW), jnp.float32),
            pltpu.SemaphoreType.DMA,
        ],
    )
    def _k(table_hbm, idx_hbm, out_hbm, idx_v, rows_v, sem):
        wid = jax.lax.axis_index("s") * 2 + jax.lax.axis_index("c")
        pltpu.sync_copy(idx_hbm.at[wid], idx_v)
        copies = [
            pltpu.async_copy(
                table_hbm.at[idx_v.at[j]],
                rows_v.at[pl.ds(j * _CHUNK, _CHUNK)],
                sem,
            )
            for j in range(nch)
        ]
        for c in copies:
            c.wait()
        pltpu.sync_copy(rows_v, out_hbm.at[pl.ds(wid * b_per_w, b_per_w)])

    return _k(W_pad, idx3)


def kernel(tokens, W):
    B, K, D = tokens.shape
    flat = tokens.reshape(-1, D)
    idx, loss_sum = _dist_argmin(flat, W)
    W_pad = jnp.pad(W, ((0, 0), (0, _ROW - D)))
    q = _gather_rows(W_pad, idx)[:, :D]
    q_st = flat + (q - flat)
    vq_loss = (1.0 + COMMITMENT_COST) * loss_sum / (flat.shape[0] * D)
    return (q_st.reshape(B, K, D), vq_loss, idx.reshape(B, K))


# speedup vs baseline: 1.1852x; 1.1852x over previous
import functools

import jax
import jax.numpy as jnp
from jax.experimental import pallas as pl
from jax.experimental.pallas import tpu as pltpu
from jax.experimental.pallas import tpu_sc as plsc

NUM_EMBEDDINGS = 1024
EMBEDDING_DIM = 64
COMMITMENT_COST = 0.25


_BM = 1024


def _dist_argmin_body(x_ref, w_ref, idx_ref, loss_ref):
    x = x_ref[...]
    w = w_ref[...]
    m = jax.lax.dot_general(x, w, (((1,), (1,)), ((), ())),
                            preferred_element_type=jnp.float32)
    x2 = jnp.sum(x * x, axis=1, keepdims=True)
    w2 = jnp.sum(w * w, axis=1)
    dist = (x2 - 2.0 * m) + w2[None, :]
    minval = jnp.min(dist, axis=1, keepdims=True)
    iota = jax.lax.broadcasted_iota(jnp.int32, dist.shape, 1).astype(jnp.float32)
    cand = jnp.where(dist == minval, iota, jnp.float32(NUM_EMBEDDINGS))
    idxf = jnp.min(cand, axis=1, keepdims=True)
    idx_ref[...] = idxf.astype(jnp.int32)

    @pl.when(pl.program_id(0) == 0)
    def _init():
        loss_ref[...] = jnp.zeros_like(loss_ref)

    loss_ref[...] += jnp.sum(minval).reshape(1, 1)


def _dist_argmin(flat, W):
    n = flat.shape[0]
    grid = (n // _BM,)
    idx, loss = pl.pallas_call(
        _dist_argmin_body,
        grid=grid,
        in_specs=[
            pl.BlockSpec((_BM, EMBEDDING_DIM), lambda i: (i, 0)),
            pl.BlockSpec((NUM_EMBEDDINGS, EMBEDDING_DIM), lambda i: (0, 0)),
        ],
        out_specs=[
            pl.BlockSpec((_BM, 1), lambda i: (i, 0)),
            pl.BlockSpec((1, 1), lambda i: (0, 0)),
        ],
        out_shape=[
            jax.ShapeDtypeStruct((n, 1), jnp.int32),
            jax.ShapeDtypeStruct((1, 1), jnp.float32),
        ],
    )(flat, W)
    return idx[:, 0], loss[0, 0]


_NW = 32
_CHUNK = 128
_ROW = 128


def _gather_rows(W_pad, idx):
    n = idx.shape[0]
    b_per_w = n // _NW
    nch = b_per_w // _CHUNK
    idx3 = idx.reshape(_NW, nch, _CHUNK)
    mesh = plsc.VectorSubcoreMesh(core_axis_name="c", subcore_axis_name="s")

    @functools.partial(
        pl.kernel,
        mesh=mesh,
        out_type=jax.ShapeDtypeStruct((n, _ROW), jnp.float32),
        scratch_types=[
            pltpu.VMEM((nch, _CHUNK), jnp.int32),
            pltpu.VMEM((b_per_w, _ROW), jnp.float32),
            pltpu.SemaphoreType.DMA,
        ],
    )
    def _k(table_hbm, idx_hbm, out_hbm, idx_v, rows_v, sem):
        wid = jax.lax.axis_index("s") * 2 + jax.lax.axis_index("c")
        pltpu.sync_copy(idx_hbm.at[wid], idx_v)
        copies = [
            pltpu.async_copy(
                table_hbm.at[idx_v.at[j]],
                rows_v.at[pl.ds(j * _CHUNK, _CHUNK)],
                sem,
            )
            for j in range(nch)
        ]
        for c in copies:
            c.wait()
        pltpu.sync_copy(rows_v, out_hbm.at[pl.ds(wid * b_per_w, b_per_w)])

    return _k(W_pad, idx3)


def kernel(tokens, W):
    B, K, D = tokens.shape
    flat = tokens.reshape(-1, D)
    idx, loss_sum = _dist_argmin(flat, W)
    W_pad = jnp.pad(W, ((0, 0), (0, _ROW - D)))
    q = _gather_rows(W_pad, idx)[:, :D]
    q_st = flat + (q - flat)
    vq_loss = (1.0 + COMMITMENT_COST) * loss_sum / (flat.shape[0] * D)
    return (q_st.reshape(B, K, D), vq_loss, idx.reshape(B, K))


# speedup vs baseline: 1.4481x; 1.2218x over previous
import functools

import jax
import jax.numpy as jnp
from jax.experimental import pallas as pl
from jax.experimental.pallas import tpu as pltpu
from jax.experimental.pallas import tpu_sc as plsc

NUM_EMBEDDINGS = 1024
EMBEDDING_DIM = 64
COMMITMENT_COST = 0.25


_BM = 1024


def _dist_argmin_body_diag(x_ref, w_ref, idx_ref, loss_ref, qst_ref):
    x = x_ref[...]
    w = w_ref[...]
    m = jax.lax.dot_general(x, w, (((1,), (1,)), ((), ())),
                            preferred_element_type=jnp.float32)
    x2 = jnp.sum(x * x, axis=1, keepdims=True)
    w2 = jnp.sum(w * w, axis=1)
    dist = (x2 - 2.0 * m) + w2[None, :]
    minval = jnp.min(dist, axis=1, keepdims=True)
    iota = jax.lax.broadcasted_iota(jnp.int32, dist.shape, 1).astype(jnp.float32)
    cand = jnp.where(dist == minval, iota, jnp.float32(NUM_EMBEDDINGS))
    idxf = jnp.min(cand, axis=1, keepdims=True)
    idx_ref[...] = idxf.astype(jnp.int32)
    onehot = jnp.where(iota == idxf, jnp.float32(1.0), jnp.float32(0.0))
    q = jax.lax.dot_general(onehot, w, (((1,), (0,)), ((), ())),
                            preferred_element_type=jnp.float32)
    qst_ref[...] = x + (q - x)

    @pl.when(pl.program_id(0) == 0)
    def _init():
        loss_ref[...] = jnp.zeros_like(loss_ref)

    loss_ref[...] += jnp.sum(minval).reshape(1, 1)


def _dist_argmin_diag(flat, W):
    n = flat.shape[0]
    grid = (n // _BM,)
    idx, loss, qst = pl.pallas_call(
        _dist_argmin_body_diag,
        grid=grid,
        in_specs=[
            pl.BlockSpec((_BM, EMBEDDING_DIM), lambda i: (i, 0)),
            pl.BlockSpec((NUM_EMBEDDINGS, EMBEDDING_DIM), lambda i: (0, 0)),
        ],
        out_specs=[
            pl.BlockSpec((_BM, 1), lambda i: (i, 0)),
            pl.BlockSpec((1, 1), lambda i: (0, 0)),
            pl.BlockSpec((_BM, EMBEDDING_DIM), lambda i: (i, 0)),
        ],
        out_shape=[
            jax.ShapeDtypeStruct((n, 1), jnp.int32),
            jax.ShapeDtypeStruct((1, 1), jnp.float32),
            jax.ShapeDtypeStruct((n, EMBEDDING_DIM), jnp.float32),
        ],
    )(flat, W)
    return idx[:, 0], loss[0, 0], qst


def _dist_argmin_body(x_ref, w_ref, idx_ref, loss_ref):
    x = x_ref[...]
    w = w_ref[...]
    m = jax.lax.dot_general(x, w, (((1,), (1,)), ((), ())),
                            preferred_element_type=jnp.float32)
    x2 = jnp.sum(x * x, axis=1, keepdims=True)
    w2 = jnp.sum(w * w, axis=1)
    dist = (x2 - 2.0 * m) + w2[None, :]
    minval = jnp.min(dist, axis=1, keepdims=True)
    iota = jax.lax.broadcasted_iota(jnp.int32, dist.shape, 1).astype(jnp.float32)
    cand = jnp.where(dist == minval, iota, jnp.float32(NUM_EMBEDDINGS))
    idxf = jnp.min(cand, axis=1, keepdims=True)
    idx_ref[...] = idxf.astype(jnp.int32)

    @pl.when(pl.program_id(0) == 0)
    def _init():
        loss_ref[...] = jnp.zeros_like(loss_ref)

    loss_ref[...] += jnp.sum(minval).reshape(1, 1)


def _dist_argmin(flat, W):
    n = flat.shape[0]
    grid = (n // _BM,)
    idx, loss = pl.pallas_call(
        _dist_argmin_body,
        grid=grid,
        in_specs=[
            pl.BlockSpec((_BM, EMBEDDING_DIM), lambda i: (i, 0)),
            pl.BlockSpec((NUM_EMBEDDINGS, EMBEDDING_DIM), lambda i: (0, 0)),
        ],
        out_specs=[
            pl.BlockSpec((_BM, 1), lambda i: (i, 0)),
            pl.BlockSpec((1, 1), lambda i: (0, 0)),
        ],
        out_shape=[
            jax.ShapeDtypeStruct((n, 1), jnp.int32),
            jax.ShapeDtypeStruct((1, 1), jnp.float32),
        ],
    )(flat, W)
    return idx[:, 0], loss[0, 0]


_NW = 32
_CHUNK = 128
_ROW = 128


def _gather_rows(W_pad, idx):
    n = idx.shape[0]
    b_per_w = n // _NW
    nch = b_per_w // _CHUNK
    idx3 = idx.reshape(_NW, nch, _CHUNK)
    mesh = plsc.VectorSubcoreMesh(core_axis_name="c", subcore_axis_name="s")

    @functools.partial(
        pl.kernel,
        mesh=mesh,
        out_type=jax.ShapeDtypeStruct((n, _ROW), jnp.float32),
        scratch_types=[
            pltpu.VMEM((nch, _CHUNK), jnp.int32),
            pltpu.VMEM((b_per_w, _ROW), jnp.float32),
            pltpu.SemaphoreType.DMA,
        ],
    )
    def _k(table_hbm, idx_hbm, out_hbm, idx_v, rows_v, sem):
        wid = jax.lax.axis_index("s") * 2 + jax.lax.axis_index("c")
        pltpu.sync_copy(idx_hbm.at[wid], idx_v)
        copies = [
            pltpu.async_copy(
                table_hbm.at[idx_v.at[j]],
                rows_v.at[pl.ds(j * _CHUNK, _CHUNK)],
                sem,
            )
            for j in range(nch)
        ]
        for c in copies:
            c.wait()
        pltpu.sync_copy(rows_v, out_hbm.at[pl.ds(wid * b_per_w, b_per_w)])

    return _k(W_pad, idx3)


def kernel(tokens, W):
    B, K, D = tokens.shape
    flat = tokens.reshape(-1, D)
    idx, loss_sum, q_st = _dist_argmin_diag(flat, W)
    vq_loss = (1.0 + COMMITMENT_COST) * loss_sum / (flat.shape[0] * D)
    return (q_st.reshape(B, K, D), vq_loss, idx.reshape(B, K))
